# Initial kernel scaffold; baseline (speedup 1.0000x reference)
import jax
import jax.numpy as jnp
from jax.experimental import pallas as pl


def kernel(x, edge_index, edge_attr, W1, b1, g1, bt1, W2, b2, g2, bt2, fcW, fcb, fc1W, fc1b):
    raise NotImplementedError("write your pallas kernel here")


# speedup vs baseline: 1.1724x; 1.1724x over previous
import jax
import jax.numpy as jnp
from jax.experimental import pallas as pl
from jax.experimental.pallas import tpu as pltpu

_EPS = 1e-5


def _col_stats(h):
    s = jnp.sum(h, axis=0, keepdims=True)
    ss = jnp.sum(h * h, axis=0, keepdims=True)
    return jnp.concatenate(
        [s, ss, jnp.zeros((6, h.shape[1]), jnp.float32)], axis=0)


def _bn_coeffs(st_ref, g_ref, bt_ref, n):
    mean = st_ref[0:1, :] * (1.0 / n)
    var = st_ref[1:2, :] * (1.0 / n) - mean * mean
    scale = g_ref[...] * jax.lax.rsqrt(var + _EPS)
    shift = bt_ref[...] - mean * scale
    return scale, shift


def _fused_body(x_ref, w1_ref, b1_ref, g1_ref, bt1_ref, w2_ref, b2_ref,
                g2_ref, bt2_ref, w3_ref, b3_ref, w4_ref, b4_ref,
                out_ref, hbuf, st1, st2, *, n, r):
    p = pl.program_id(0)
    i = pl.program_id(1)
    rows = pl.ds(i * r, r)

    @pl.when(p == 0)
    def _phase0():
        h1 = jnp.dot(x_ref[...], w1_ref[...],
                     preferred_element_type=jnp.float32) + b1_ref[...]
        hbuf[rows, :] = h1

        @pl.when(i == 0)
        def _():
            st1[...] = jnp.zeros_like(st1)

        st1[...] += _col_stats(h1)

    @pl.when(p == 1)
    def _phase1():
        scale, shift = _bn_coeffs(st1, g1_ref, bt1_ref, n)
        a = jnp.maximum(hbuf[rows, :] * scale + shift, 0.0)
        h2 = jnp.dot(a, w2_ref[...],
                     preferred_element_type=jnp.float32) + b2_ref[...]
        hbuf[rows, :] = h2

        @pl.when(i == 0)
        def _():
            st2[...] = jnp.zeros_like(st2)

        st2[...] += _col_stats(h2)

    @pl.when(p == 2)
    def _phase2():
        scale, shift = _bn_coeffs(st2, g2_ref, bt2_ref, n)
        b = jnp.maximum(hbuf[rows, :] * scale + shift, 0.0)
        c = jnp.dot(b, w3_ref[...], preferred_element_type=jnp.float32)
        c = jnp.maximum(c + b3_ref[...], 0.0)
        o = jnp.dot(c, w4_ref[...], preferred_element_type=jnp.float32)
        out_ref[...] = o + b4_ref[...]


def kernel(x, edge_index, edge_attr, W1, b1, g1, bt1, W2, b2, g2, bt2,
           fcW, fcb, fc1W, fc1b):
    del edge_index, edge_attr
    n, f = x.shape
    h = W1.shape[1]
    h3 = fcW.shape[1]
    o = fc1W.shape[1]
    r = 400 if n % 400 == 0 else n
    grid = (3, n // r)

    import functools
    row2d = lambda v: v.reshape(1, -1)
    const = lambda shape: pl.BlockSpec(shape, lambda p, i: (0, 0))

    out = pl.pallas_call(
        functools.partial(_fused_body, n=n, r=r),
        grid=grid,
        in_specs=[
            pl.BlockSpec((r, f), lambda p, i: (jnp.where(p == 0, i, 0), 0)),
            const((f, h)),
            const((1, h)),
            const((1, h)),
            const((1, h)),
            const((h, h)),
            const((1, h)),
            const((1, h)),
            const((1, h)),
            const((h, h3)),
            const((1, h3)),
            const((h3, o)),
            const((1, o)),
        ],
        out_specs=pl.BlockSpec((r, o), lambda p, i: (i, 0)),
        out_shape=jax.ShapeDtypeStruct((n, o), jnp.float32),
        scratch_shapes=[
            pltpu.VMEM((n, h), jnp.float32),
            pltpu.VMEM((8, h), jnp.float32),
            pltpu.VMEM((8, h), jnp.float32),
        ],
        compiler_params=pltpu.CompilerParams(
            dimension_semantics=("arbitrary", "arbitrary")),
    )(x, W1, row2d(b1), row2d(g1), row2d(bt1), W2, row2d(b2), row2d(g2),
      row2d(bt2), fcW, row2d(fcb), fc1W, row2d(fc1b))

    return out


# speedup vs baseline: 1.5003x; 1.2797x over previous
import jax
import jax.numpy as jnp
from jax.experimental import pallas as pl
from jax.experimental.pallas import tpu as pltpu

_EPS = 1e-5


def _col_stats(h):
    s = jnp.sum(h, axis=0, keepdims=True)
    ss = jnp.sum(h * h, axis=0, keepdims=True)
    return jnp.concatenate(
        [s, ss, jnp.zeros((6, h.shape[1]), jnp.float32)], axis=0)


def _bn_coeffs(st_ref, g_ref, bt_ref, n):
    mean = st_ref[0:1, :] * (1.0 / n)
    var = st_ref[1:2, :] * (1.0 / n) - mean * mean
    scale = g_ref[...] * jax.lax.rsqrt(var + _EPS)
    shift = bt_ref[...] - mean * scale
    return scale, shift


def _fused_body(x_ref, w1_ref, b1_ref, g1_ref, bt1_ref, w2_ref, b2_ref,
                g2_ref, bt2_ref, w3_ref, b3_ref, w4_ref, b4_ref,
                out_ref, hbuf, st1, st2, *, n, r):
    p = pl.program_id(0)
    i = pl.program_id(1)
    rows = pl.ds(i * r, r)

    @pl.when(p == 0)
    def _phase0():
        h1 = jnp.dot(x_ref[...], w1_ref[...],
                     preferred_element_type=jnp.float32) + b1_ref[...]
        hbuf[rows, :] = h1

        @pl.when(i == 0)
        def _():
            st1[...] = jnp.zeros_like(st1)

        st1[...] += _col_stats(h1)

    @pl.when(p == 1)
    def _phase1():
        scale, shift = _bn_coeffs(st1, g1_ref, bt1_ref, n)
        a = jnp.maximum(hbuf[rows, :] * scale + shift, 0.0)
        h2 = jnp.dot(a, w2_ref[...],
                     preferred_element_type=jnp.float32) + b2_ref[...]
        hbuf[rows, :] = h2

        @pl.when(i == 0)
        def _():
            st2[...] = jnp.zeros_like(st2)

        st2[...] += _col_stats(h2)

    @pl.when(p == 2)
    def _phase2():
        scale, shift = _bn_coeffs(st2, g2_ref, bt2_ref, n)
        b = jnp.maximum(hbuf[rows, :] * scale + shift, 0.0)
        c = jnp.dot(b, w3_ref[...], preferred_element_type=jnp.float32)
        c = jnp.maximum(c + b3_ref[...], 0.0)
        o = jnp.dot(c, w4_ref[...], preferred_element_type=jnp.float32)
        out_ref[...] = o + b4_ref[...]


def kernel(x, edge_index, edge_attr, W1, b1, g1, bt1, W2, b2, g2, bt2,
           fcW, fcb, fc1W, fc1b):
    del edge_index, edge_attr
    n, f = x.shape
    h = W1.shape[1]
    h3 = fcW.shape[1]
    o = fc1W.shape[1]
    r = 1000 if n % 1000 == 0 else n
    grid = (3, n // r)

    import functools
    row2d = lambda v: v.reshape(1, -1)
    const = lambda shape: pl.BlockSpec(shape, lambda p, i: (0, 0))

    out = pl.pallas_call(
        functools.partial(_fused_body, n=n, r=r),
        grid=grid,
        in_specs=[
            pl.BlockSpec((r, f), lambda p, i: (jnp.where(p == 0, i, 0), 0)),
            const((f, h)),
            const((1, h)),
            const((1, h)),
            const((1, h)),
            const((h, h)),
            const((1, h)),
            const((1, h)),
            const((1, h)),
            const((h, h3)),
            const((1, h3)),
            const((h3, o)),
            const((1, o)),
        ],
        out_specs=pl.BlockSpec((r, o), lambda p, i: (i, 0)),
        out_shape=jax.ShapeDtypeStruct((n, o), jnp.float32),
        scratch_shapes=[
            pltpu.VMEM((n, h), jnp.float32),
            pltpu.VMEM((8, h), jnp.float32),
            pltpu.VMEM((8, h), jnp.float32),
        ],
        compiler_params=pltpu.CompilerParams(
            dimension_semantics=("arbitrary", "arbitrary")),
    )(x, W1, row2d(b1), row2d(g1), row2d(bt1), W2, row2d(b2), row2d(g2),
      row2d(bt2), fcW, row2d(fcb), fc1W, row2d(fc1b))

    return out
